# Initial kernel scaffold; baseline (speedup 1.0000x reference)
import jax
import jax.numpy as jnp
from jax.experimental import pallas as pl


def kernel(x, conv_w, conv_b):
    raise NotImplementedError("write your pallas kernel here")


# speedup vs baseline: 1.9031x; 1.9031x over previous
import jax
import jax.numpy as jnp
from jax.experimental import pallas as pl
from jax.experimental.pallas import tpu as pltpu


def _conv_body(xodd_ref, xeven_ref, w_ref, b_ref, o_ref):
    lout = o_ref.shape[2]
    xodd = xodd_ref[0]
    xeven = xeven_ref[0]
    acc = jnp.dot(w_ref[0], xodd[:, 0:lout],
                  preferred_element_type=jnp.float32)
    acc = acc + jnp.dot(w_ref[1], xeven,
                        preferred_element_type=jnp.float32)
    acc = acc + jnp.dot(w_ref[2], xodd[:, 1:lout + 1],
                        preferred_element_type=jnp.float32)
    o_ref[0] = (acc + b_ref[...]).astype(o_ref.dtype)


def kernel(x, conv_w, conv_b):
    B, Cin, L = x.shape
    Cout = conv_w.shape[0]
    assert conv_w.shape == (Cout, Cin, 3)
    assert L % 2 == 0 and L >= 4
    Lout = L // 2

    xb = x.astype(jnp.bfloat16)
    x_odd = jnp.concatenate([xb[..., 1:2], xb[..., 1::2]], axis=-1)
    x_even = xb[..., 0::2]

    w_k = jnp.transpose(conv_w, (2, 0, 1)).astype(jnp.bfloat16)
    b2 = conv_b.reshape(Cout, 1).astype(jnp.float32)

    return pl.pallas_call(
        _conv_body,
        out_shape=jax.ShapeDtypeStruct((B, Cout, Lout), x.dtype),
        grid=(B,),
        in_specs=[
            pl.BlockSpec((1, Cin, Lout + 1), lambda b: (b, 0, 0)),
            pl.BlockSpec((1, Cin, Lout), lambda b: (b, 0, 0)),
            pl.BlockSpec((3, Cout, Cin), lambda b: (0, 0, 0)),
            pl.BlockSpec((Cout, 1), lambda b: (0, 0)),
        ],
        out_specs=pl.BlockSpec((1, Cout, Lout), lambda b: (b, 0, 0)),
        compiler_params=pltpu.CompilerParams(
            dimension_semantics=("parallel",)),
    )(x_odd, x_even, w_k, b2)


# speedup vs baseline: 20.0810x; 10.5516x over previous
import jax
import jax.numpy as jnp
from jax.experimental import pallas as pl
from jax.experimental.pallas import tpu as pltpu


def _conv_body(x_ref, p_ref, w_ref, b_ref, o_ref):
    lout = o_ref.shape[2]
    xt = x_ref[0].astype(jnp.bfloat16)
    s = jnp.dot(xt, p_ref[...], preferred_element_type=jnp.float32)
    s = s.astype(jnp.bfloat16)
    even = s[:, 0:lout]
    odd = s[:, lout:2 * lout]
    odd_prev = jnp.concatenate([odd[:, :1], odd[:, :-1]], axis=1)
    acc = jnp.dot(w_ref[0], odd_prev, preferred_element_type=jnp.float32)
    acc = acc + jnp.dot(w_ref[1], even, preferred_element_type=jnp.float32)
    acc = acc + jnp.dot(w_ref[2], odd, preferred_element_type=jnp.float32)
    o_ref[0] = (acc + b_ref[...]).astype(o_ref.dtype)


def kernel(x, conv_w, conv_b):
    B, Cin, L = x.shape
    Cout = conv_w.shape[0]
    assert conv_w.shape == (Cout, Cin, 3)
    assert L % 2 == 0 and L >= 4
    Lout = L // 2

    m = jax.lax.broadcasted_iota(jnp.int32, (L, 2 * Lout), 0)
    l_col = jax.lax.broadcasted_iota(jnp.int32, (L, 2 * Lout), 1)
    sel = (m == jnp.where(l_col < Lout, 2 * l_col, 2 * (l_col - Lout) + 1))
    p_sel = sel.astype(jnp.bfloat16)

    w_k = jnp.transpose(conv_w, (2, 0, 1)).astype(jnp.bfloat16)
    b2 = conv_b.reshape(Cout, 1).astype(jnp.float32)

    return pl.pallas_call(
        _conv_body,
        out_shape=jax.ShapeDtypeStruct((B, Cout, Lout), x.dtype),
        grid=(B,),
        in_specs=[
            pl.BlockSpec((1, Cin, L), lambda b: (b, 0, 0)),
            pl.BlockSpec((L, 2 * Lout), lambda b: (0, 0)),
            pl.BlockSpec((3, Cout, Cin), lambda b: (0, 0, 0)),
            pl.BlockSpec((Cout, 1), lambda b: (0, 0)),
        ],
        out_specs=pl.BlockSpec((1, Cout, Lout), lambda b: (b, 0, 0)),
        compiler_params=pltpu.CompilerParams(
            dimension_semantics=("parallel",),
            vmem_limit_bytes=64 * 1024 * 1024),
    )(x, p_sel, w_k, b2)
